# baseline (speedup 1.0000x reference)
import functools

import jax
import jax.numpy as jnp
from jax import lax
from jax.experimental import pallas as pl
from jax.experimental.pallas import tpu as pltpu
from jax.experimental.pallas import tpu_sc as plsc

_N_CH = 256
_N_SRC = 8192
_N_TOK = 2048
_B = 8
_NC, _NS = 2, 16
_NW = _NC * _NS
_RPW = _N_TOK // _NW
_HPW = _RPW // 2
_TI = 1024
_KC = 4


def _bias_mm_body(l_ref, a_ref, out_ref, acc_ref):
    k = pl.program_id(0)

    @pl.when(k == 0)
    def _():
        acc_ref[...] = jnp.zeros_like(acc_ref)

    acc_ref[...] += lax.dot_general(
        l_ref[...], l_ref[...],
        dimension_numbers=(((1,), (1,)), ((), ())),
        preferred_element_type=jnp.float32,
    )

    @pl.when(k == _KC - 1)
    def _():
        out_ref[...] = a_ref[0, 0] * acc_ref[...]


def _scaled_bias(L_row, alpha):
    return pl.pallas_call(
        _bias_mm_body,
        grid=(_KC,),
        in_specs=[
            pl.BlockSpec((_N_CH, _N_SRC // _KC), lambda k: (0, k)),
            pl.BlockSpec((1, 1), lambda k: (0, 0)),
        ],
        out_specs=pl.BlockSpec((_N_CH, _N_CH), lambda k: (0, 0)),
        out_shape=jax.ShapeDtypeStruct((_N_CH, _N_CH), jnp.float32),
        scratch_shapes=[pltpu.VMEM((_N_CH, _N_CH), jnp.float32)],
    )(L_row, alpha.reshape(1, 1))


def _sc_row_gather(bias, idx):
    mesh = plsc.VectorSubcoreMesh(
        core_axis_name="c", subcore_axis_name="s",
        num_cores=_NC, num_subcores=_NS,
    )

    @functools.partial(
        pl.kernel,
        out_type=jax.ShapeDtypeStruct((_N_TOK, _N_CH), jnp.float32),
        mesh=mesh,
        compiler_params=pltpu.CompilerParams(
            use_tc_tiling_on_sc=True, needs_layout_passes=False),
        scratch_types=[
            pltpu.VMEM((_HPW,), jnp.int32),
            pltpu.VMEM((_HPW,), jnp.int32),
            pltpu.VMEM((_HPW, _N_CH), jnp.float32),
            pltpu.VMEM((_HPW, _N_CH), jnp.float32),
            pltpu.SemaphoreType.DMA,
            pltpu.SemaphoreType.DMA,
            pltpu.SemaphoreType.DMA,
            pltpu.SemaphoreType.DMA,
        ],
    )
    def k(bias_hbm, idx_hbm, out_hbm, idx0_v, idx1_v, rows0_v, rows1_v,
          g0_sem, g1_sem, w0_sem, w1_sem):
        wid = lax.axis_index("s") * _NC + lax.axis_index("c")
        base = wid * _RPW
        pltpu.sync_copy(idx_hbm.at[pl.ds(base, _HPW)], idx0_v)
        g0 = pltpu.async_copy(bias_hbm.at[idx0_v], rows0_v, g0_sem)
        pltpu.sync_copy(idx_hbm.at[pl.ds(base + _HPW, _HPW)], idx1_v)
        g1 = pltpu.async_copy(bias_hbm.at[idx1_v], rows1_v, g1_sem)
        g0.wait()
        w0 = pltpu.async_copy(rows0_v, out_hbm.at[pl.ds(base, _HPW)], w0_sem)
        g1.wait()
        w1 = pltpu.async_copy(
            rows1_v, out_hbm.at[pl.ds(base + _HPW, _HPW)], w1_sem)
        w0.wait()
        w1.wait()

    return k(bias, idx)


def _add_body(attn_ref, r_ref, idx_ref, mask_ref, out_ref, fb_ref):
    @pl.when(pl.program_id(1) == 0)
    def _():
        onehot_t = (idx_ref[...][None, :] == lax.broadcasted_iota(
            jnp.int32, (_N_CH, _N_TOK), 0)).astype(jnp.float32)
        fb_ref[...] = lax.dot_general(
            r_ref[...], onehot_t,
            dimension_numbers=(((1,), (0,)), ((), ())),
            preferred_element_type=jnp.float32,
        )

    b = jnp.where(mask_ref[...] != 0, fb_ref[...], 0.0)
    out_ref[...] = attn_ref[...] + b[None]


def _fused_add(attn_logits, rows, idx1d, mask_i8):
    return pl.pallas_call(
        _add_body,
        grid=(_N_TOK // _TI, _B),
        in_specs=[
            pl.BlockSpec((1, _TI, _N_TOK), lambda i, b: (b, i, 0)),
            pl.BlockSpec((_TI, _N_CH), lambda i, b: (i, 0)),
            pl.BlockSpec((_N_TOK,), lambda i, b: (0,)),
            pl.BlockSpec((_TI, _N_TOK), lambda i, b: (i, 0)),
        ],
        out_specs=pl.BlockSpec((1, _TI, _N_TOK), lambda i, b: (b, i, 0)),
        out_shape=jax.ShapeDtypeStruct((_B, _N_TOK, _N_TOK), jnp.float32),
        scratch_shapes=[pltpu.VMEM((_TI, _N_TOK), jnp.float32)],
    )(attn_logits, rows, idx1d, mask_i8)


def kernel(attn_logits, L_row, alpha, ch_tok_mask, ch_indices):
    bias = _scaled_bias(L_row, alpha)
    rows = _sc_row_gather(bias, ch_indices)
    return _fused_add(attn_logits, rows, ch_indices,
                      ch_tok_mask.view(jnp.uint8))


# speedup vs baseline: 1.0090x; 1.0090x over previous
import functools

import jax
import jax.numpy as jnp
from jax import lax
from jax.experimental import pallas as pl
from jax.experimental.pallas import tpu as pltpu
from jax.experimental.pallas import tpu_sc as plsc

_N_CH = 256
_N_SRC = 8192
_N_TOK = 2048
_B = 8
_NC, _NS = 2, 16
_NW = _NC * _NS
_RPW = _N_TOK // _NW
_HPW = _RPW // 2
_TI = 1024
_KC = 2


def _bias_mm_body(l_ref, a_ref, out_ref, acc_ref):
    k = pl.program_id(0)

    @pl.when(k == 0)
    def _():
        acc_ref[...] = jnp.zeros_like(acc_ref)

    acc_ref[...] += lax.dot_general(
        l_ref[...], l_ref[...],
        dimension_numbers=(((1,), (1,)), ((), ())),
        preferred_element_type=jnp.float32,
    )

    @pl.when(k == _KC - 1)
    def _():
        out_ref[...] = a_ref[0, 0] * acc_ref[...]


def _scaled_bias(L_row, alpha):
    return pl.pallas_call(
        _bias_mm_body,
        grid=(_KC,),
        in_specs=[
            pl.BlockSpec((_N_CH, _N_SRC // _KC), lambda k: (0, k)),
            pl.BlockSpec((1, 1), lambda k: (0, 0)),
        ],
        out_specs=pl.BlockSpec((_N_CH, _N_CH), lambda k: (0, 0)),
        out_shape=jax.ShapeDtypeStruct((_N_CH, _N_CH), jnp.float32),
        scratch_shapes=[pltpu.VMEM((_N_CH, _N_CH), jnp.float32)],
    )(L_row, alpha.reshape(1, 1))


def _sc_row_gather(bias, idx):
    mesh = plsc.VectorSubcoreMesh(
        core_axis_name="c", subcore_axis_name="s",
        num_cores=_NC, num_subcores=_NS,
    )

    @functools.partial(
        pl.kernel,
        out_type=jax.ShapeDtypeStruct((_N_TOK, _N_CH), jnp.float32),
        mesh=mesh,
        compiler_params=pltpu.CompilerParams(
            use_tc_tiling_on_sc=True, needs_layout_passes=False),
        scratch_types=[
            pltpu.VMEM((_HPW,), jnp.int32),
            pltpu.VMEM((_HPW,), jnp.int32),
            pltpu.VMEM((_HPW, _N_CH), jnp.float32),
            pltpu.VMEM((_HPW, _N_CH), jnp.float32),
            pltpu.SemaphoreType.DMA,
            pltpu.SemaphoreType.DMA,
            pltpu.SemaphoreType.DMA,
            pltpu.SemaphoreType.DMA,
        ],
    )
    def k(bias_hbm, idx_hbm, out_hbm, idx0_v, idx1_v, rows0_v, rows1_v,
          g0_sem, g1_sem, w0_sem, w1_sem):
        wid = lax.axis_index("s") * _NC + lax.axis_index("c")
        base = wid * _RPW
        pltpu.sync_copy(idx_hbm.at[pl.ds(base, _HPW)], idx0_v)
        g0 = pltpu.async_copy(bias_hbm.at[idx0_v], rows0_v, g0_sem)
        pltpu.sync_copy(idx_hbm.at[pl.ds(base + _HPW, _HPW)], idx1_v)
        g1 = pltpu.async_copy(bias_hbm.at[idx1_v], rows1_v, g1_sem)
        g0.wait()
        w0 = pltpu.async_copy(rows0_v, out_hbm.at[pl.ds(base, _HPW)], w0_sem)
        g1.wait()
        w1 = pltpu.async_copy(
            rows1_v, out_hbm.at[pl.ds(base + _HPW, _HPW)], w1_sem)
        w0.wait()
        w1.wait()

    return k(bias, idx)


def _add_body(attn_ref, r_ref, idx_ref, mask_ref, out_ref, fb_ref):
    @pl.when(pl.program_id(1) == 0)
    def _():
        onehot_t = (idx_ref[...][None, :] == lax.broadcasted_iota(
            jnp.int32, (_N_CH, _N_TOK), 0)).astype(jnp.float32)
        fb_ref[...] = lax.dot_general(
            r_ref[...], onehot_t,
            dimension_numbers=(((1,), (0,)), ((), ())),
            preferred_element_type=jnp.float32,
        )

    b = jnp.where(mask_ref[...] != 0, fb_ref[...], 0.0)
    out_ref[...] = attn_ref[...] + b[None]


def _fused_add(attn_logits, rows, idx1d, mask_i8):
    return pl.pallas_call(
        _add_body,
        grid=(_N_TOK // _TI, _B),
        in_specs=[
            pl.BlockSpec((1, _TI, _N_TOK), lambda i, b: (b, i, 0)),
            pl.BlockSpec((_TI, _N_CH), lambda i, b: (i, 0)),
            pl.BlockSpec((_N_TOK,), lambda i, b: (0,)),
            pl.BlockSpec((_TI, _N_TOK), lambda i, b: (i, 0)),
        ],
        out_specs=pl.BlockSpec((1, _TI, _N_TOK), lambda i, b: (b, i, 0)),
        out_shape=jax.ShapeDtypeStruct((_B, _N_TOK, _N_TOK), jnp.float32),
        scratch_shapes=[pltpu.VMEM((_TI, _N_TOK), jnp.float32)],
    )(attn_logits, rows, idx1d, mask_i8)


def kernel(attn_logits, L_row, alpha, ch_tok_mask, ch_indices):
    bias = _scaled_bias(L_row, alpha)
    rows = _sc_row_gather(bias, ch_indices)
    return _fused_add(attn_logits, rows, ch_indices,
                      ch_tok_mask.view(jnp.uint8))
